# baseline (speedup 1.0000x reference)
import functools

import jax
import jax.numpy as jnp
from jax import lax
from jax.experimental import pallas as pl
from jax.experimental.pallas import tpu as pltpu
from jax.experimental.pallas import tpu_sc as plsc

_NC = 2
_NS = 16
_LANES = 16
_CHUNK = 80
_PASS_CHUNKS = 32


def _sc_body(n_passes, rows_per_tile, nvec,
             x_hbm, src_hbm, dst_hbm, adj_hbm, partial_hbm,
             src_all, dst_all, adj_all, g_a, g_b, acc, sem_a, sem_b):
    c = lax.axis_index("c")
    s = lax.axis_index("s")
    wid = c * _NS + s

    zero = jnp.zeros((_LANES,), jnp.float32)
    for r in range(8):
        for k in range(nvec):
            g_a[r, pl.ds(k * _LANES, _LANES)] = zero

    def zcopy(b, carry):
        pltpu.sync_copy(g_a.at[pl.ds(0, 8)],
                        acc.at[pl.ds(s * rows_per_tile + b * 8, 8)])
        return carry

    lax.fori_loop(0, rows_per_tile // 8, zcopy, 0)
    plsc.subcore_barrier()

    chunk_base = wid * (n_passes * _PASS_CHUNKS)

    def compute_scale(buf, ci):
        def group(g, ecarry):
            a16 = adj_all[ci, pl.ds(g * _LANES, _LANES)]
            for j in range(_LANES):
                av = jnp.full((_LANES,), a16[j], jnp.float32)
                row = g * _LANES + j
                for k in range(nvec):
                    sl = pl.ds(k * _LANES, _LANES)
                    buf[row, sl] = buf[row, sl] * av
            return ecarry

        lax.fori_loop(0, _CHUNK // _LANES, group, 0)

    def pass_body(p, carry):
        row0 = chunk_base + p * _PASS_CHUNKS
        pltpu.sync_copy(src_hbm.at[pl.ds(row0, _PASS_CHUNKS)], src_all)
        pltpu.sync_copy(dst_hbm.at[pl.ds(row0, _PASS_CHUNKS)], dst_all)
        pltpu.sync_copy(adj_hbm.at[pl.ds(row0, _PASS_CHUNKS)], adj_all)
        pltpu.async_copy(x_hbm.at[src_all.at[0]], g_a, sem_a)
        pltpu.async_copy(x_hbm.at[src_all.at[1]], g_b, sem_b)

        def process(buf, sem, ci):
            pltpu.make_async_copy(x_hbm.at[pl.ds(0, _CHUNK)], buf, sem).wait()
            compute_scale(buf, ci)
            pltpu.sync_copy(buf, acc.at[dst_all.at[ci]], add=True)

            @pl.when(ci + 2 < _PASS_CHUNKS)
            def _():
                pltpu.async_copy(x_hbm.at[src_all.at[ci + 2]], buf, sem)

        def chunk_body(ci, carry2):
            @pl.when(ci % 2 == 0)
            def _():
                process(g_a, sem_a, ci)

            @pl.when(ci % 2 == 1)
            def _():
                process(g_b, sem_b, ci)

            return carry2

        lax.fori_loop(0, _PASS_CHUNKS, chunk_body, 0)
        return carry

    lax.fori_loop(0, n_passes, pass_body, 0)
    plsc.subcore_barrier()

    r0 = s * rows_per_tile
    pltpu.sync_copy(acc.at[pl.ds(r0, rows_per_tile)],
                    partial_hbm.at[c, pl.ds(r0, rows_per_tile)])


@jax.jit
def _sc_spmm(x, src2, dst2, adj2):
    n, d = x.shape
    n_chunks_total = src2.shape[0]
    nw = _NC * _NS
    per_worker_chunks = n_chunks_total // nw
    n_passes = per_worker_chunks // _PASS_CHUNKS
    align = _NS * 8
    n_pad = ((n + align - 1) // align) * align
    rows_per_tile = n_pad // _NS

    mesh = plsc.VectorSubcoreMesh(core_axis_name="c", subcore_axis_name="s")
    body = functools.partial(_sc_body, n_passes, rows_per_tile, d // _LANES)
    f = pl.kernel(
        body,
        out_type=jax.ShapeDtypeStruct((_NC, n_pad, d), jnp.float32),
        mesh=mesh,
        scratch_types=[
            pltpu.VMEM((_PASS_CHUNKS, _CHUNK), jnp.int32),
            pltpu.VMEM((_PASS_CHUNKS, _CHUNK), jnp.int32),
            pltpu.VMEM((_PASS_CHUNKS, _CHUNK), jnp.float32),
            pltpu.VMEM((_CHUNK, d), jnp.float32),
            pltpu.VMEM((_CHUNK, d), jnp.float32),
            pltpu.VMEM_SHARED((n_pad, d), jnp.float32),
            pltpu.SemaphoreType.DMA,
            pltpu.SemaphoreType.DMA,
        ],
    )
    return f(x, src2, dst2, adj2)


def _combine_body(p_ref, w_ref, o_ref):
    o_ref[...] = (p_ref[0] + p_ref[1]) * w_ref[...]


def _combine(partial, w2d, n):
    _, n_pad, d = partial.shape
    blk = 1000 if n % 1000 == 0 else n
    grid_r = n // blk
    return pl.pallas_call(
        _combine_body,
        grid=(grid_r,),
        in_specs=[
            pl.BlockSpec((_NC, blk, d), lambda i: (0, i, 0)),
            pl.BlockSpec((1, d), lambda i: (0, 0)),
        ],
        out_specs=pl.BlockSpec((blk, d), lambda i: (i, 0)),
        out_shape=jax.ShapeDtypeStruct((n, d), jnp.float32),
    )(partial, w2d)


def kernel(x, edge_index, adj_values, W):
    n, d = x.shape
    e = adj_values.shape[0]
    unit = _CHUNK * _NC * _NS * 8
    e_pad = ((e + unit - 1) // unit) * unit
    pad = e_pad - e
    dst = jnp.pad(edge_index[0], (0, pad)).reshape(e_pad // _CHUNK, _CHUNK)
    src = jnp.pad(edge_index[1], (0, pad)).reshape(e_pad // _CHUNK, _CHUNK)
    adj2 = jnp.pad(adj_values, (0, pad)).reshape(e_pad // _CHUNK, _CHUNK)
    partial = _sc_spmm(x, src, dst, adj2)
    return _combine(partial, W.reshape(1, d), n)
